# baseline (speedup 1.0000x reference)
import numpy as np

import functools

import jax
import jax.numpy as jnp
from jax import lax
from jax.experimental import pallas as pl
from jax.experimental.pallas import tpu as pltpu
from jax.experimental.pallas import tpu_sc as plsc

C = 96
H = 224
W = 224
YA = 16
XA = 16
CB = 32


def _bicubic_crop_weights(in_size: int, out_full: int, crop_start: int,
                          crop_len: int) -> np.ndarray:
    inv_scale = in_size / out_full
    sample_f = (np.arange(out_full, dtype=np.float64) + 0.5) * inv_scale - 0.5
    x = np.abs(sample_f[None, :] - np.arange(in_size, dtype=np.float64)[:, None])
    out = ((1.5 * x - 2.5) * x) * x + 1.0
    out = np.where(x >= 1.0, ((-0.5 * x + 2.5) * x - 4.0) * x + 2.0, out)
    w = np.where(x >= 2.0, 0.0, out)
    w = w / w.sum(axis=0, keepdims=True)
    w = w[:, crop_start:crop_start + crop_len]
    return np.ascontiguousarray(w.T.astype(np.float32))


_WY = _bicubic_crop_weights(4, 3 * H, H // 2, H)
_WXT = _bicubic_crop_weights(4, 3 * W, W // 2, W).T.copy()
_NPIX = float(H * W)


def _sc_gather(mt_hbm, st_hbm, idx_hbm, gm_hbm, gs_hbm,
               idx_v, rowsm_v, rowss_v, sem):
    wid = lax.axis_index("s") * 2 + lax.axis_index("c")

    @pl.when(wid == 0)
    def _():
        pltpu.sync_copy(idx_hbm, idx_v)
        pltpu.async_copy(mt_hbm.at[idx_v], rowsm_v, sem).wait()
        pltpu.sync_copy(rowsm_v, gm_hbm)
        pltpu.async_copy(st_hbm.at[idx_v], rowss_v, sem).wait()
        pltpu.sync_copy(rowss_v, gs_hbm)


def _sc_gather_call(mt, st, idx):
    mesh = plsc.VectorSubcoreMesh(core_axis_name="c", subcore_axis_name="s")
    sds = jax.ShapeDtypeStruct((16, 128), jnp.float32)
    return pl.kernel(
        _sc_gather, mesh=mesh,
        out_type=(sds, sds),
        scratch_types=[
            pltpu.VMEM((16,), jnp.int32),
            pltpu.VMEM((16, 128), jnp.float32),
            pltpu.VMEM((16, 128), jnp.float32),
            pltpu.SemaphoreType.DMA,
        ],
    )(mt, st, idx)


def _body(ya_ref, xa_ref, py_ref, px_ref, x_ref, mt_ref, st_ref,
          wy_ref, wxt_ref, wv_ref, bv_ref, out_ref, gma_ref, gsa_ref):
    pre = x_ref[1]
    s1 = jnp.sum(pre, axis=(1, 2), keepdims=True)
    s2 = jnp.sum(pre * pre, axis=(1, 2), keepdims=True)
    pm3 = s1 * (1.0 / _NPIX)
    var3 = (s2 - pm3 * s1) * (1.0 / (_NPIX - 1.0))
    pinv3 = 1.0 / jnp.sqrt(var3)

    wv = wv_ref[0]
    bv = bv_ref[0]
    psw = pinv3 * wv
    out_ref[1] = pre * psw + (bv - pm3 * psw)

    sy = jnp.clip(ya_ref[0, 0], 0, YA - 1)
    sx = jnp.clip(xa_ref[0, 0], 0, XA - 1)
    py = py_ref[0, 0]
    px = px_ref[0, 0]
    gma_ref[...] = mt_ref[:, :C].T
    gsa_ref[...] = st_ref[:, :C].T

    c0 = pl.program_id(0) * CB
    gm = gma_ref[pl.ds(c0, CB), :]
    gs = gsa_ref[pl.ds(c0, CB), :]
    gi = 1.0 / gs

    kk = jax.lax.broadcasted_iota(jnp.int32, (1, 16), 1)
    rowk = jnp.clip(sy + kk // 4 - 1, 0, YA - 1)
    colk = jnp.clip(sx + kk % 4 - 1, 0, XA - 1)
    m16 = ((rowk == py) & (colk == px)).astype(jnp.float32)

    pm2 = pm3[:, 0, :]
    pinv2 = pinv3[:, 0, :]
    wv2 = wv[:, 0, :]
    gm = gm * (1.0 - m16) + pm2 * m16
    gi = gi * (1.0 - m16) + pinv2 * m16

    wxt = wxt_ref[...]
    wy = wy_ref[...]
    um_rows = []
    us_rows = []
    for i in range(4):
        gm_i = gm[:, i * 4:(i + 1) * 4]
        gi_i = gi[:, i * 4:(i + 1) * 4]
        um_rows.append(jnp.dot(gm_i, wxt, preferred_element_type=jnp.float32)[:, None, :])
        us_rows.append((jnp.dot(gi_i, wxt, preferred_element_type=jnp.float32) * wv2)[:, None, :])
    um = jnp.concatenate(um_rows, axis=1)
    us = jnp.concatenate(us_rows, axis=1)

    for c in range(CB):
        mm = jnp.dot(wy, um[c], preferred_element_type=jnp.float32)
        ssw = jnp.dot(wy, us[c], preferred_element_type=jnp.float32)
        out_ref[0, c] = (x_ref[0, c] - mm) * ssw + bv[c]


def kernel(x, mean_table, std_table, weight, bias, y_anchor, x_anchor,
           pre_y1_anchor, pre_x1_anchor):
    ya = jnp.asarray(y_anchor, jnp.int32).reshape(1, 1)
    xa = jnp.asarray(x_anchor, jnp.int32).reshape(1, 1)
    py = jnp.asarray(pre_y1_anchor, jnp.int32).reshape(1, 1)
    px = jnp.asarray(pre_x1_anchor, jnp.int32).reshape(1, 1)
    mt = mean_table.reshape(YA * XA, C)
    st = std_table.reshape(YA * XA, C)
    kk = jnp.arange(16, dtype=jnp.int32)
    sy0 = jnp.clip(jnp.asarray(y_anchor, jnp.int32), 0, YA - 1)
    sx0 = jnp.clip(jnp.asarray(x_anchor, jnp.int32), 0, XA - 1)
    idx = (jnp.clip(sy0 + kk // 4 - 1, 0, YA - 1) * XA +
           jnp.clip(sx0 + kk % 4 - 1, 0, XA - 1))
    mtp = jnp.pad(mt, ((0, 0), (0, 128 - C)))
    stp = jnp.pad(st, ((0, 0), (0, 128 - C)))
    gm16, gs16 = _sc_gather_call(mtp, stp, idx)
    wv = weight
    bv = bias

    nblk = C // CB
    smem = pl.BlockSpec(memory_space=pltpu.SMEM)
    out = pl.pallas_call(
        _body,
        grid=(nblk,),
        in_specs=[
            smem, smem, smem, smem,
            pl.BlockSpec((2, CB, H, W), lambda cb: (0, cb, 0, 0)),
            pl.BlockSpec((16, 128), lambda cb: (0, 0)),
            pl.BlockSpec((16, 128), lambda cb: (0, 0)),
            pl.BlockSpec((H, 4), lambda cb: (0, 0)),
            pl.BlockSpec((4, W), lambda cb: (0, 0)),
            pl.BlockSpec((1, CB, 1, 1), lambda cb: (0, cb, 0, 0)),
            pl.BlockSpec((1, CB, 1, 1), lambda cb: (0, cb, 0, 0)),
        ],
        out_specs=pl.BlockSpec((2, CB, H, W), lambda cb: (0, cb, 0, 0)),
        out_shape=jax.ShapeDtypeStruct((2, C, H, W), jnp.float32),
        scratch_shapes=[pltpu.VMEM((C, 16), jnp.float32),
                        pltpu.VMEM((C, 16), jnp.float32)],
        compiler_params=pltpu.CompilerParams(
            dimension_semantics=("arbitrary",),
            vmem_limit_bytes=110 * 1024 * 1024),
    )(ya, xa, py, px, x, gm16, gs16, jnp.asarray(_WY), jnp.asarray(_WXT), wv, bv)
    return out


# speedup vs baseline: 1.5120x; 1.5120x over previous
import numpy as np

import jax
import jax.numpy as jnp
from jax.experimental import pallas as pl
from jax.experimental.pallas import tpu as pltpu

C = 96
H = 224
W = 224
YA = 16
XA = 16
CB = 32


def _bicubic_crop_weights(in_size: int, out_full: int, crop_start: int,
                          crop_len: int) -> np.ndarray:
    inv_scale = in_size / out_full
    sample_f = (np.arange(out_full, dtype=np.float64) + 0.5) * inv_scale - 0.5
    x = np.abs(sample_f[None, :] - np.arange(in_size, dtype=np.float64)[:, None])
    out = ((1.5 * x - 2.5) * x) * x + 1.0
    out = np.where(x >= 1.0, ((-0.5 * x + 2.5) * x - 4.0) * x + 2.0, out)
    w = np.where(x >= 2.0, 0.0, out)
    w = w / w.sum(axis=0, keepdims=True)
    w = w[:, crop_start:crop_start + crop_len]
    return np.ascontiguousarray(w.T.astype(np.float32))


_WY = _bicubic_crop_weights(4, 3 * H, H // 2, H)
_WXT = _bicubic_crop_weights(4, 3 * W, W // 2, W).T.copy()
_NPIX = float(H * W)


def _body(ya_ref, xa_ref, py_ref, px_ref, x_ref, mt_ref, st_ref,
          wy_ref, wxt_ref, wv_ref, bv_ref, out_ref, gma_ref, gsa_ref):
    pre = x_ref[1]
    s1 = jnp.sum(pre, axis=(1, 2), keepdims=True)
    s2 = jnp.sum(pre * pre, axis=(1, 2), keepdims=True)
    pm3 = s1 * (1.0 / _NPIX)
    var3 = (s2 - pm3 * s1) * (1.0 / (_NPIX - 1.0))
    pinv3 = 1.0 / jnp.sqrt(var3)

    wv = wv_ref[0]
    bv = bv_ref[0]
    psw = pinv3 * wv
    out_ref[1] = pre * psw + (bv - pm3 * psw)

    sy = jnp.clip(ya_ref[0, 0], 0, YA - 1)
    sx = jnp.clip(xa_ref[0, 0], 0, XA - 1)
    py = py_ref[0, 0]
    px = px_ref[0, 0]
    ry = [jnp.clip(sy + (i - 1), 0, YA - 1) for i in range(4)]
    rx = [jnp.clip(sx + (j - 1), 0, XA - 1) for j in range(4)]
    gm_rows = []
    gs_rows = []
    for i in range(4):
        for j in range(4):
            p = ry[i] * XA + rx[j]
            gm_rows.append(mt_ref[pl.ds(p, 1), :])
            gs_rows.append(st_ref[pl.ds(p, 1), :])
    gma_ref[...] = jnp.concatenate(gm_rows, axis=0).T
    gsa_ref[...] = jnp.concatenate(gs_rows, axis=0).T

    c0 = pl.program_id(0) * CB
    gm = gma_ref[pl.ds(c0, CB), :]
    gs = gsa_ref[pl.ds(c0, CB), :]
    gi = 1.0 / gs

    kk = jax.lax.broadcasted_iota(jnp.int32, (1, 16), 1)
    rowk = jnp.clip(sy + kk // 4 - 1, 0, YA - 1)
    colk = jnp.clip(sx + kk % 4 - 1, 0, XA - 1)
    m16 = ((rowk == py) & (colk == px)).astype(jnp.float32)

    pm2 = pm3[:, 0, :]
    pinv2 = pinv3[:, 0, :]
    wv2 = wv[:, 0, :]
    gm = gm * (1.0 - m16) + pm2 * m16
    gi = gi * (1.0 - m16) + pinv2 * m16

    wxt = wxt_ref[...]
    wy = wy_ref[...]
    um_rows = []
    us_rows = []
    for i in range(4):
        gm_i = gm[:, i * 4:(i + 1) * 4]
        gi_i = gi[:, i * 4:(i + 1) * 4]
        um_rows.append(jnp.dot(gm_i, wxt, preferred_element_type=jnp.float32)[:, None, :])
        us_rows.append((jnp.dot(gi_i, wxt, preferred_element_type=jnp.float32) * wv2)[:, None, :])
    um = jnp.concatenate(um_rows, axis=1)
    us = jnp.concatenate(us_rows, axis=1)

    for c in range(CB):
        mm = jnp.dot(wy, um[c], preferred_element_type=jnp.float32)
        ssw = jnp.dot(wy, us[c], preferred_element_type=jnp.float32)
        out_ref[0, c] = (x_ref[0, c] - mm) * ssw + bv[c]


def kernel(x, mean_table, std_table, weight, bias, y_anchor, x_anchor,
           pre_y1_anchor, pre_x1_anchor):
    ya = jnp.asarray(y_anchor, jnp.int32).reshape(1, 1)
    xa = jnp.asarray(x_anchor, jnp.int32).reshape(1, 1)
    py = jnp.asarray(pre_y1_anchor, jnp.int32).reshape(1, 1)
    px = jnp.asarray(pre_x1_anchor, jnp.int32).reshape(1, 1)
    mt = mean_table.reshape(YA * XA, C)
    st = std_table.reshape(YA * XA, C)
    wv = weight
    bv = bias

    nblk = C // CB
    smem = pl.BlockSpec(memory_space=pltpu.SMEM)
    out = pl.pallas_call(
        _body,
        grid=(nblk,),
        in_specs=[
            smem, smem, smem, smem,
            pl.BlockSpec((2, CB, H, W), lambda cb: (0, cb, 0, 0)),
            pl.BlockSpec((YA * XA, C), lambda cb: (0, 0)),
            pl.BlockSpec((YA * XA, C), lambda cb: (0, 0)),
            pl.BlockSpec((H, 4), lambda cb: (0, 0)),
            pl.BlockSpec((4, W), lambda cb: (0, 0)),
            pl.BlockSpec((1, CB, 1, 1), lambda cb: (0, cb, 0, 0)),
            pl.BlockSpec((1, CB, 1, 1), lambda cb: (0, cb, 0, 0)),
        ],
        out_specs=pl.BlockSpec((2, CB, H, W), lambda cb: (0, cb, 0, 0)),
        out_shape=jax.ShapeDtypeStruct((2, C, H, W), jnp.float32),
        scratch_shapes=[pltpu.VMEM((C, 16), jnp.float32),
                        pltpu.VMEM((C, 16), jnp.float32)],
        compiler_params=pltpu.CompilerParams(
            dimension_semantics=("arbitrary",),
            vmem_limit_bytes=110 * 1024 * 1024),
    )(ya, xa, py, px, x, mt, st, jnp.asarray(_WY), jnp.asarray(_WXT), wv, bv)
    return out
